# baseline (speedup 1.0000x reference)
import jax
import jax.numpy as jnp
from jax.experimental import pallas as pl
from jax.experimental.pallas import tpu as pltpu

_SIZE = 100000
_PAD = 0
_SMOOTHING = 0.1
_CONF = 1.0 - _SMOOTHING
_EPS = _SMOOTHING / (_SIZE - 2)

_B = 1024
_R = 8
_NRB = _B // _R


def _loss_body(t_ref, x_ref, o_ref):
    i = pl.program_id(0)
    x = x_ref[...]
    t = t_ref[0]
    col = jax.lax.broadcasted_iota(jnp.int32, (_R, _SIZE), 1)
    s = jnp.sum(x, axis=1, keepdims=True) - x[:, 0:1]
    g = jnp.sum(jnp.where(col == t, x, 0.0), axis=1, keepdims=True)
    c1 = _SMOOTHING * jnp.log(jnp.float32(_EPS)) + _CONF * jnp.log(
        jnp.float32(_CONF))
    row_term = c1 - _EPS * s + (_EPS - _CONF) * g
    partial = jnp.sum(jnp.where(t != _PAD, row_term, 0.0)) * (1.0 / _B)

    @pl.when(i == 0)
    def _init():
        o_ref[0, 0] = 0.0

    o_ref[0, 0] += partial


@jax.jit
def kernel(x, target):
    t2 = target.astype(jnp.int32).reshape(_NRB, _R, 1)
    out = pl.pallas_call(
        _loss_body,
        grid=(_NRB,),
        in_specs=[
            pl.BlockSpec((1, _R, 1), lambda i: (i, 0, 0)),
            pl.BlockSpec((_R, _SIZE), lambda i: (i, 0)),
        ],
        out_specs=pl.BlockSpec(memory_space=pltpu.SMEM),
        out_shape=jax.ShapeDtypeStruct((1, 1), jnp.float32),
    )(t2, x)
    return out[0, 0]


# speedup vs baseline: 1.1801x; 1.1801x over previous
import jax
import jax.numpy as jnp
from jax.experimental import pallas as pl
from jax.experimental.pallas import tpu as pltpu

_SIZE = 100000
_PAD = 0
_SMOOTHING = 0.1
_CONF = 1.0 - _SMOOTHING
_EPS = _SMOOTHING / (_SIZE - 2)

_B = 1024
_R = 64
_NRB = _B // _R


def _loss_body(t_ref, x_ref, o_ref):
    i = pl.program_id(0)
    x = x_ref[...]
    t = t_ref[0]
    col = jax.lax.broadcasted_iota(jnp.int32, (_R, _SIZE), 1)
    s = jnp.sum(x, axis=1, keepdims=True) - x[:, 0:1]
    g = jnp.sum(jnp.where(col == t, x, 0.0), axis=1, keepdims=True)
    c1 = _SMOOTHING * jnp.log(jnp.float32(_EPS)) + _CONF * jnp.log(
        jnp.float32(_CONF))
    row_term = c1 - _EPS * s + (_EPS - _CONF) * g
    partial = jnp.sum(jnp.where(t != _PAD, row_term, 0.0)) * (1.0 / _B)

    @pl.when(i == 0)
    def _init():
        o_ref[0, 0] = 0.0

    o_ref[0, 0] += partial


@jax.jit
def kernel(x, target):
    t2 = target.astype(jnp.int32).reshape(_NRB, _R, 1)
    out = pl.pallas_call(
        _loss_body,
        grid=(_NRB,),
        in_specs=[
            pl.BlockSpec((1, _R, 1), lambda i: (i, 0, 0)),
            pl.BlockSpec((_R, _SIZE), lambda i: (i, 0)),
        ],
        out_specs=pl.BlockSpec(memory_space=pltpu.SMEM),
        out_shape=jax.ShapeDtypeStruct((1, 1), jnp.float32),
    )(t2, x)
    return out[0, 0]


# speedup vs baseline: 1.1817x; 1.0013x over previous
import jax
import jax.numpy as jnp
from jax.experimental import pallas as pl
from jax.experimental.pallas import tpu as pltpu

_SIZE = 100000
_PAD = 0
_SMOOTHING = 0.1
_CONF = 1.0 - _SMOOTHING
_EPS = _SMOOTHING / (_SIZE - 2)

_B = 1024
_NBAND = 4
_R = 16
_BAND = _B // _NBAND
_NRB = _BAND // _R


def _band_partial(t, x):
    col = jax.lax.broadcasted_iota(jnp.int32, (_R, _SIZE), 1)
    s = jnp.sum(x, axis=1, keepdims=True) - x[:, 0:1]
    g = jnp.sum(jnp.where(col == t, x, 0.0), axis=1, keepdims=True)
    c1 = _SMOOTHING * jnp.log(jnp.float32(_EPS)) + _CONF * jnp.log(
        jnp.float32(_CONF))
    row_term = c1 - _EPS * s + (_EPS - _CONF) * g
    return jnp.sum(jnp.where(t != _PAD, row_term, 0.0))


def _loss_body(*refs):
    t_refs = refs[:_NBAND]
    x_refs = refs[_NBAND:2 * _NBAND]
    o_ref = refs[2 * _NBAND]
    i = pl.program_id(0)
    partial = 0.0
    for band in range(_NBAND):
        partial += _band_partial(t_refs[band][0], x_refs[band][...])

    @pl.when(i == 0)
    def _init():
        o_ref[0, 0] = 0.0

    o_ref[0, 0] += partial * (1.0 / _B)


@jax.jit
def kernel(x, target):
    t3 = target.astype(jnp.int32).reshape(_B // _R, _R, 1)
    t_in = [t3 for _ in range(_NBAND)]
    x_in = [x for _ in range(_NBAND)]
    t_specs = [
        pl.BlockSpec((1, _R, 1), lambda i, b=band: (b * _NRB + i, 0, 0))
        for band in range(_NBAND)
    ]
    x_specs = [
        pl.BlockSpec((_R, _SIZE), lambda i, b=band: (b * _NRB + i, 0))
        for band in range(_NBAND)
    ]
    out = pl.pallas_call(
        _loss_body,
        grid=(_NRB,),
        in_specs=t_specs + x_specs,
        out_specs=pl.BlockSpec(memory_space=pltpu.SMEM),
        out_shape=jax.ShapeDtypeStruct((1, 1), jnp.float32),
    )(*t_in, *x_in)
    return out[0, 0]
